# baseline (speedup 1.0000x reference)
import functools

import jax
import jax.numpy as jnp
import numpy as np
from jax import lax
from jax.experimental import pallas as pl
from jax.experimental.pallas import tpu as pltpu
from jax.experimental.pallas import tpu_sc as plsc

_BATCH = 16384
_N_FIELDS = 26
_EMBED_DIM = 32
_VOCAB = 100000
_N = _BATCH * _N_FIELDS
_NC = 2
_NS = 16
_NW = _NC * _NS
_PER_W = _N // _NW
_IDX_ROWS = _PER_W // 128
_PAT_ROWS = 13
_CHUNK = 1024
_GATHERS = _CHUNK // 128
_NCHUNKS = _PER_W // _CHUNK

_OFF_PATTERN = (
    (np.arange(_PAT_ROWS * 128, dtype=np.int64) % _N_FIELDS) * _VOCAB
).astype(np.int32).reshape(_PAT_ROWS, 128)

_mesh = plsc.VectorSubcoreMesh(core_axis_name="c", subcore_axis_name="s")


@functools.partial(
    pl.kernel,
    out_type=jax.ShapeDtypeStruct((_N, _EMBED_DIM), jnp.float32),
    mesh=_mesh,
    scratch_types=[
        pltpu.VMEM((_IDX_ROWS, 128), jnp.int32),
        pltpu.VMEM((_PAT_ROWS, 128), jnp.int32),
        pltpu.VMEM((_CHUNK, _EMBED_DIM), jnp.float32),
        pltpu.SemaphoreType.DMA,
    ],
    compiler_params=pltpu.CompilerParams(use_tc_tiling_on_sc=False),
)
def _embed_kernel(x_hbm, off_hbm, table_hbm, out_hbm, idx_v, off_v, buf_v, gsem):
    wid = lax.axis_index("s") * _NC + lax.axis_index("c")
    row0 = wid * _IDX_ROWS
    base = wid * _PER_W

    pltpu.sync_copy(x_hbm.at[pl.ds(row0, _IDX_ROWS)], idx_v)
    pltpu.sync_copy(off_hbm, off_v)

    def _adjust(r, carry):
        pr = r % _PAT_ROWS
        for c in range(8):
            sl = pl.ds(c * 16, 16)
            idx_v[r, sl] = idx_v[r, sl] + off_v[pr, sl]
        return carry

    lax.fori_loop(0, _IDX_ROWS, _adjust, 0)

    def _chunk(ci, carry):
        for j in range(_GATHERS):
            pltpu.async_copy(
                table_hbm.at[idx_v.at[ci * _GATHERS + j]],
                buf_v.at[pl.ds(j * 128, 128)],
                gsem,
            )
        pltpu.make_async_copy(
            table_hbm.at[pl.ds(0, _CHUNK)], buf_v, gsem
        ).wait()
        pltpu.sync_copy(buf_v, out_hbm.at[pl.ds(base + ci * _CHUNK, _CHUNK)])
        return carry

    lax.fori_loop(0, _NCHUNKS, _chunk, 0)


def kernel(x, embedding_table):
    x2 = x.reshape(_N // 128, 128)
    out = _embed_kernel(x2, jnp.asarray(_OFF_PATTERN), embedding_table)
    return out.reshape(_BATCH, _N_FIELDS, _EMBED_DIM)
